# Initial kernel scaffold; baseline (speedup 1.0000x reference)
import jax
import jax.numpy as jnp
from jax.experimental import pallas as pl


def kernel(node_ids, edge_idx, node_table, edge_table, W_lin, b_lin):
    raise NotImplementedError("write your pallas kernel here")


# speedup vs baseline: 8.3766x; 8.3766x over previous
import functools

import jax
import jax.numpy as jnp
from jax.experimental import pallas as pl
from jax.experimental.pallas import tpu as pltpu


def _linear_kernel(x_ref, w_ref, b_ref, o_ref):
    o_ref[...] = (
        jnp.dot(x_ref[...], w_ref[...], preferred_element_type=jnp.float32)
        + b_ref[...]
    )


@functools.partial(jax.jit, static_argnames=("blk",))
def _apply_linear(table, wt, b2d, blk):
    rows, dim = table.shape
    grid = (rows // blk,)
    return pl.pallas_call(
        _linear_kernel,
        grid=grid,
        in_specs=[
            pl.BlockSpec((blk, dim), lambda i: (i, 0)),
            pl.BlockSpec((dim, dim), lambda i: (0, 0)),
            pl.BlockSpec((1, dim), lambda i: (0, 0)),
        ],
        out_specs=pl.BlockSpec((blk, dim), lambda i: (i, 0)),
        out_shape=jax.ShapeDtypeStruct((rows, dim), jnp.float32),
        compiler_params=pltpu.CompilerParams(
            dimension_semantics=("arbitrary",),
        ),
    )(table, wt, b2d)


def kernel(node_ids, edge_idx, node_table, edge_table, W_lin, b_lin):
    wt = W_lin.T
    b2d = b_lin.reshape(1, -1)
    x = _apply_linear(node_table, wt, b2d, blk=2000)
    e = _apply_linear(edge_table, wt, b2d, blk=2000)
    return (x, e)


# speedup vs baseline: 8.3856x; 1.0011x over previous
import functools

import jax
import jax.numpy as jnp
from jax.experimental import pallas as pl
from jax.experimental.pallas import tpu as pltpu


def _linear_kernel(x_ref, w_ref, b_ref, o_ref):
    o_ref[...] = (
        jnp.dot(x_ref[...], w_ref[...], preferred_element_type=jnp.float32)
        + b_ref[...]
    )


@functools.partial(jax.jit, static_argnames=("blk",))
def _apply_linear(table, wt, b2d, blk):
    rows, dim = table.shape
    grid = (rows // blk,)
    return pl.pallas_call(
        _linear_kernel,
        grid=grid,
        in_specs=[
            pl.BlockSpec((blk, dim), lambda i: (i, 0)),
            pl.BlockSpec((dim, dim), lambda i: (0, 0)),
            pl.BlockSpec((1, dim), lambda i: (0, 0)),
        ],
        out_specs=pl.BlockSpec((blk, dim), lambda i: (i, 0)),
        out_shape=jax.ShapeDtypeStruct((rows, dim), jnp.float32),
        compiler_params=pltpu.CompilerParams(
            dimension_semantics=("parallel",),
        ),
    )(table, wt, b2d)


def kernel(node_ids, edge_idx, node_table, edge_table, W_lin, b_lin):
    wt = W_lin.T
    b2d = b_lin.reshape(1, -1)
    x = _apply_linear(node_table, wt, b2d, blk=2000)
    e = _apply_linear(edge_table, wt, b2d, blk=2000)
    return (x, e)


# speedup vs baseline: 13.3843x; 1.5961x over previous
import functools

import jax
import jax.numpy as jnp
from jax.experimental import pallas as pl
from jax.experimental.pallas import tpu as pltpu


def _linear_kernel(x_ref, w_ref, b_ref, o_ref):
    o_ref[...] = (
        jnp.dot(x_ref[...], w_ref[...], preferred_element_type=jnp.float32)
        + b_ref[...]
    )


@functools.partial(jax.jit, static_argnames=("blk",))
def _apply_linear(table, wt, b2d, blk):
    rows, dim = table.shape
    grid = (rows // blk,)
    return pl.pallas_call(
        _linear_kernel,
        grid=grid,
        in_specs=[
            pl.BlockSpec((blk, dim), lambda i: (i, 0)),
            pl.BlockSpec((dim, dim), lambda i: (0, 0)),
            pl.BlockSpec((1, dim), lambda i: (0, 0)),
        ],
        out_specs=pl.BlockSpec((blk, dim), lambda i: (i, 0)),
        out_shape=jax.ShapeDtypeStruct((rows, dim), jnp.float32),
        compiler_params=pltpu.CompilerParams(
            dimension_semantics=("parallel",),
        ),
    )(table, wt, b2d)


def kernel(node_ids, edge_idx, node_table, edge_table, W_lin, b_lin):
    wt = W_lin.T
    b2d = b_lin.reshape(1, -1)
    x = _apply_linear(node_table, wt, b2d, blk=2000)
    e = _apply_linear(edge_table, wt, b2d, blk=8000)
    return (x, e)


# speedup vs baseline: 13.9726x; 1.0440x over previous
import functools

import jax
import jax.numpy as jnp
from jax.experimental import pallas as pl
from jax.experimental.pallas import tpu as pltpu


def _linear_kernel(x_ref, w_ref, b_ref, o_ref):
    o_ref[...] = (
        jnp.dot(x_ref[...], w_ref[...], preferred_element_type=jnp.float32)
        + b_ref[...]
    )


@functools.partial(jax.jit, static_argnames=("blk",))
def _apply_linear(table, wt, b2d, blk):
    rows, dim = table.shape
    grid = (rows // blk,)
    return pl.pallas_call(
        _linear_kernel,
        grid=grid,
        in_specs=[
            pl.BlockSpec((blk, dim), lambda i: (i, 0)),
            pl.BlockSpec((dim, dim), lambda i: (0, 0)),
            pl.BlockSpec((1, dim), lambda i: (0, 0)),
        ],
        out_specs=pl.BlockSpec((blk, dim), lambda i: (i, 0)),
        out_shape=jax.ShapeDtypeStruct((rows, dim), jnp.float32),
        compiler_params=pltpu.CompilerParams(
            dimension_semantics=("parallel",),
        ),
    )(table, wt, b2d)


def kernel(node_ids, edge_idx, node_table, edge_table, W_lin, b_lin):
    wt = W_lin.T
    b2d = b_lin.reshape(1, -1)
    x = _apply_linear(node_table, wt, b2d, blk=5000)
    e = _apply_linear(edge_table, wt, b2d, blk=16000)
    return (x, e)
